# baseline (speedup 1.0000x reference)
import functools

import jax
import jax.numpy as jnp
from jax import lax
from jax.experimental import pallas as pl
from jax.experimental.pallas import tpu as pltpu
from jax.experimental.pallas import tpu_sc as plsc

_LAMBDA = 0.15
_B = 16384
_D = 64
_L = 16
_NC = 2
_NS = 16
_NW = _NC * _NS
_NCLS = 100000
_CW = 512
_NCHUNKS = 195
_SPBASE = _NCHUNKS * _CW
_MAXJ = 7
_NCAP = 960
_CCAP = 256
_LBits = 17
_LMask = (1 << _LBits) - 1

_mesh = plsc.VectorSubcoreMesh(core_axis_name="c", subcore_axis_name="s")


@functools.partial(
    pl.kernel,
    mesh=_mesh,
    compiler_params=pltpu.CompilerParams(needs_layout_passes=False),
    out_type=jax.ShapeDtypeStruct((_NW * _L,), jnp.float32),
    scratch_types=[
        pltpu.VMEM((_B,), jnp.int32),
        pltpu.VMEM((_NCAP + _L,), jnp.int32),
        pltpu.VMEM((_CCAP + _L,), jnp.int32),
        pltpu.VMEM((_D, _CW), jnp.float32),
        pltpu.VMEM(((_NCAP + _L) * _D,), jnp.float32),
        pltpu.VMEM(((_NCLS - _SPBASE) * _D,), jnp.float32),
        pltpu.VMEM((_L,), jnp.float32),
        pltpu.SemaphoreType.DMA,
        pltpu.SemaphoreType.DMA,
        pltpu.SemaphoreType.DMA,
    ],
)
def _sc_loss(feat_hbm, table_hbm, lab_hbm, tail_hbm, out_hbm,
             labels_v, mypk_v, cpk_v, slab_v, frows_v, tail_v,
             acc_v, sem_l, sem_f, sem_g):
    cid = lax.axis_index("c")
    sid = lax.axis_index("s")
    wid = sid * _NC + cid
    start = 6 * wid + jnp.minimum(wid, 3)
    count = jnp.where(wid < 3, 7, 6)
    lo = start * _CW
    hi = jnp.where(wid == _NW - 1, _NCLS, (start + count) * _CW)

    pltpu.async_copy(lab_hbm.at[pl.ds(0, _B)], labels_v, sem_l).wait()

    lane = lax.iota(jnp.int32, _L)

    def scan_body(v, w):
        pks, pcs = [], []
        for u in range(4):
            lbl = labels_v[pl.ds((v * 4 + u) * _L, _L)]
            pk = (((v * 4 + u) * _L + lane) << _LBits) | lbl
            m = (lbl >= lo) & (lbl < hi)
            pks.append((pk, m))
            pcs.append(plsc.all_reduce_population_count(m)[0])
        for u in range(4):
            plsc.store_compressed(
                mypk_v.at[pl.ds(jnp.minimum(w, _NCAP), _L)],
                pks[u][0], mask=pks[u][1])
            w = w + pcs[u]
        return w

    n = lax.fori_loop(0, _B // (4 * _L), scan_body, jnp.int32(0))
    n = jnp.minimum(n, _NCAP)
    ngr_n = (n + _L - 1) // _L
    ngr_n2 = (n + 2 * _L - 1) // (2 * _L)

    def rescan(cb_lo, cb_hi):
        def body(v, wc):
            pcs, pms = [], []
            for u in range(2):
                pk = mypk_v[pl.ds((v * 2 + u) * _L, _L)]
                lbl = pk & _LMask
                k = (v * 2 + u) * _L + lane
                m = (k < n) & (lbl >= cb_lo) & (lbl < cb_hi)
                pms.append(((k << _LBits) | lbl, m))
                pcs.append(plsc.all_reduce_population_count(m)[0])
            for u in range(2):
                plsc.store_compressed(
                    cpk_v.at[pl.ds(jnp.minimum(wc, _CCAP), _L)],
                    pms[u][0], mask=pms[u][1])
                wc = wc + pcs[u]
            return wc

        m_c = lax.fori_loop(0, ngr_n2, body, jnp.int32(0))
        return jnp.minimum(m_c, _CCAP)

    def fetch_all_rows(ngr):
        def frow(g, carry):
            pk = mypk_v[pl.ds(g * _L, _L)]
            ids = pk >> _LBits
            for u in range(_L):
                iid = jnp.clip(ids[u], 0, _B - 1)
                slot = jnp.minimum(g * _L + u, _NCAP - 1)
                pltpu.async_copy(
                    feat_hbm.at[pl.ds(iid * _D, _D)],
                    frows_v.at[pl.ds(slot * _D, _D)], sem_f)
            return carry

        lax.fori_loop(0, ngr, frow, jnp.int32(0))

        def fdrain(g, carry):
            pltpu.make_async_copy(
                feat_hbm.at[pl.ds(0, _L * _D)],
                frows_v.at[pl.ds(0, _L * _D)], sem_f).wait()
            return carry

        lax.fori_loop(0, ngr, fdrain, jnp.int32(0))

    def distances(m_c, cb, sw, acc, src_v, row_major):
        def grp(g, acc2):
            k = g * _L + lane
            valid = k < m_c
            pk = cpk_v[pl.ds(g * _L, _L)]
            cls = jnp.clip((pk & _LMask) - cb, 0, sw - 1)
            kc = jnp.clip(pk >> _LBits, 0, _NCAP - 1)
            s0 = jnp.zeros((_L,), jnp.float32)
            s1 = jnp.zeros((_L,), jnp.float32)
            s2 = jnp.zeros((_L,), jnp.float32)
            s3 = jnp.zeros((_L,), jnp.float32)
            parts = [s0, s1, s2, s3]
            for d in range(_D):
                dv = jnp.full((_L,), d, jnp.int32)
                if row_major:
                    p = plsc.load_gather(src_v, [cls * _D + dv])
                else:
                    p = plsc.load_gather(src_v, [dv, cls])
                f = plsc.load_gather(frows_v, [kc * _D + dv])
                df = f - p
                parts[d % 4] = parts[d % 4] + df * df
            x = (parts[0] + parts[1]) + (parts[2] + parts[3])
            x = jnp.where(valid, x, jnp.float32(0))
            i = lax.bitcast_convert_type(x, jnp.int32)
            i = jnp.int32(0x5F3759DF) - (i >> 1)
            y = lax.bitcast_convert_type(i, jnp.float32)
            for _ in range(3):
                y = y * (jnp.float32(1.5) - jnp.float32(0.5) * x * y * y)
            return acc2 + x * y

        return lax.fori_loop(0, (m_c + _L - 1) // _L, grp, acc)

    fetch_all_rows(ngr_n)

    def chunk_body(j, acc):
        cidj = start + jnp.minimum(j, count - 1)
        validj = j < count
        cb = pl.multiple_of(cidj * _CW, _CW)
        slab_cp = pltpu.async_copy(
            table_hbm.at[:, pl.ds(cb, _CW)], slab_v, sem_g)
        mb = jnp.where(validj, cb, jnp.int32(1 << 27))
        m_c = rescan(mb, mb + _CW)
        slab_cp.wait()
        return distances(m_c, cb, _CW, acc, slab_v, False)

    acc = lax.fori_loop(0, _MAXJ, chunk_body,
                        jnp.zeros((_L,), jnp.float32))

    sp_cp = pltpu.async_copy(tail_hbm.at[pl.ds(0, (_NCLS - _SPBASE) * _D)],
                             tail_v, sem_g)
    m_sp = rescan(jnp.int32(_SPBASE), jnp.int32(_NCLS))
    sp_cp.wait()
    acc = distances(m_sp, jnp.int32(_SPBASE), _NCLS - _SPBASE,
                    acc, tail_v, True)

    acc_v[...] = acc
    pltpu.sync_copy(acc_v, out_hbm.at[pl.ds(wid * _L, _L)])


def kernel(feature_prototypes, prototypes, labels):
    tail = lax.slice(prototypes, (_SPBASE, 0), (_NCLS, _D)).reshape(-1)
    partials = _sc_loss(feature_prototypes.reshape(-1), prototypes.T,
                        labels.astype(jnp.int32), tail)
    return (_LAMBDA / _B) * jnp.sum(partials)


# speedup vs baseline: 1.3342x; 1.3342x over previous
import functools

import jax
import jax.numpy as jnp
from jax import lax
from jax.experimental import pallas as pl
from jax.experimental.pallas import tpu as pltpu
from jax.experimental.pallas import tpu_sc as plsc

_LAMBDA = 0.15
_B = 16384
_D = 64
_L = 16
_NC = 2
_NS = 16
_NW = _NC * _NS
_BPW = _B // _NW
_CH = 256
_NCH = _BPW // _CH
_GROUPS = _CH // _L

_mesh = plsc.VectorSubcoreMesh(core_axis_name="c", subcore_axis_name="s")


@functools.partial(
    pl.kernel,
    mesh=_mesh,
    out_type=jax.ShapeDtypeStruct((_NW * _L,), jnp.float32),
    scratch_types=[
        pltpu.VMEM((_BPW,), jnp.int32),
        pltpu.VMEM((_CH, _D), jnp.float32),
        pltpu.VMEM((_CH, _D), jnp.float32),
        pltpu.VMEM((_L,), jnp.float32),
        pltpu.SemaphoreType.DMA,
        pltpu.SemaphoreType.DMA,
        pltpu.SemaphoreType.DMA,
    ],
)
def _sc_loss(feat_hbm, table_hbm, lab_hbm, out_hbm,
             lab_v, rows_v, feat_v, acc_v, sem_l, sem_f, sem_g):
    cid = lax.axis_index("c")
    sid = lax.axis_index("s")
    wid = sid * _NC + cid
    base = wid * _BPW

    pltpu.async_copy(lab_hbm.at[pl.ds(base, _BPW)], lab_v, sem_l).wait()

    lane = lax.iota(jnp.int32, _L)
    lane_masks = [lane == jnp.int32(rr) for rr in range(_L)]
    shuffles = [jnp.bitwise_xor(lane, jnp.int32(k)) for k in (8, 4, 2, 1)]
    _dnums = lax.GatherDimensionNumbers(
        offset_dims=(), collapsed_slice_dims=(0,), start_index_map=(0,))

    def hsum_splat(v):
        for perm in shuffles:
            v = v + lax.gather(
                v, perm[:, None], dimension_numbers=_dnums,
                slice_sizes=(1,),
                mode=lax.GatherScatterMode.PROMISE_IN_BOUNDS)
        return v

    def make_group_body(ch):
        def group_body(g, acc):
            row0 = g * _L
            tot = jnp.zeros((_L,), jnp.float32)
            for rr in range(_L):
                r = row0 + rr
                parts = []
                for c in range(_D // _L):
                    f = feat_v[r, pl.ds(c * _L, _L)]
                    p = rows_v[r, pl.ds(c * _L, _L)]
                    df = f - p
                    parts.append(df * df)
                sq = (parts[0] + parts[1]) + (parts[2] + parts[3])
                tot = jnp.where(lane_masks[rr], hsum_splat(sq), tot)
            x = tot
            i = lax.bitcast_convert_type(x, jnp.int32)
            i = jnp.int32(0x5F3759DF) - (i >> 1)
            y = lax.bitcast_convert_type(i, jnp.float32)
            for _ in range(3):
                y = y * (jnp.float32(1.5) - jnp.float32(0.5) * x * y * y)
            return acc + x * y
        return group_body

    acc = jnp.zeros((_L,), jnp.float32)
    for ch in range(_NCH):
        cbase = base + ch * _CH
        feat_cp = pltpu.async_copy(
            feat_hbm.at[pl.ds(cbase, _CH)], feat_v, sem_f)

        def issue(i, carry, _ch=ch):
            lvec = lab_v[pl.ds(_ch * _CH + i * _L, _L)]
            for u in range(_L):
                pltpu.async_copy(table_hbm.at[pl.ds(lvec[u], 1)],
                                 rows_v.at[pl.ds(i * _L + u, 1)], sem_g)
            return carry

        lax.fori_loop(0, _CH // _L, issue, jnp.int32(0))
        pltpu.make_async_copy(
            table_hbm.at[pl.ds(0, _CH)], rows_v, sem_g).wait()
        feat_cp.wait()
        acc = lax.fori_loop(0, _GROUPS, make_group_body(ch), acc)

    acc_v[...] = acc
    pltpu.sync_copy(acc_v, out_hbm.at[pl.ds(wid * _L, _L)])


def kernel(feature_prototypes, prototypes, labels):
    partials = _sc_loss(feature_prototypes, prototypes,
                        labels.astype(jnp.int32))
    return (_LAMBDA / _B) * jnp.sum(partials)
